# baseline (speedup 1.0000x reference)
import functools

import jax
import jax.numpy as jnp
from jax import lax
from jax.experimental import pallas as pl
from jax.experimental.pallas import tpu as pltpu
from jax.experimental.pallas import tpu_sc as plsc

N = 10000
NP = 10240
E = 320000
D = 128

NC = 2
NS = 16
NW = NC * NS
EPT = E // NW
CH = 80
NCH = EPT // CH
KD = 2
NSETD = NCH // KD
KS = 2
NGS = NCH // (2 * KS)
RPT = NP // NS

_mesh = plsc.VectorSubcoreMesh(core_axis_name="c", subcore_axis_name="s")


@functools.partial(
    pl.kernel,
    out_type=jax.ShapeDtypeStruct((NC, NP, D), jnp.float32),
    mesh=_mesh,
    scratch_types=(
        [pltpu.VMEM((CH, D), jnp.float32)] * 2
        + [pltpu.VMEM((CH,), jnp.int32)] * (4 * KD)
        + [pltpu.VMEM((CH,), jnp.int32)] * (4 * KD)
        + [
            pltpu.VMEM_SHARED((NP, D), jnp.float32),
        ]
        + [pltpu.SemaphoreType.DMA] * 4
        + [
            pltpu.SemaphoreType.DMA,
            pltpu.SemaphoreType.DMA,
            pltpu.SemaphoreType.DMA,
        ]
    ),
)
def _deg_kernel(src_hbm, dst_hbm, onehot_hbm, zeros_hbm, out_hbm,
                ohs_v, ohd_v, *rest):
    sidx = rest[0:4 * KD]
    didx = rest[4 * KD:8 * KD]
    deg_sh = rest[8 * KD]
    semi = rest[8 * KD + 1:8 * KD + 5]
    sems_a, sems_b, semz = rest[8 * KD + 5:]
    sems = (sems_a, sems_b)
    c = lax.axis_index("c")
    s = lax.axis_index("s")
    base = (s * NC + c) * EPT
    rows = pl.ds(s * RPT, RPT)
    zcp = pltpu.async_copy(zeros_hbm, deg_sh.at[rows], semz)
    pltpu.sync_copy(onehot_hbm.at[0], ohs_v)
    pltpu.sync_copy(onehot_hbm.at[1], ohd_v)

    def fire_idx(off, j):
        for k in range(KD):
            sl = pl.ds(off + k * CH, CH)
            pltpu.async_copy(src_hbm.at[sl], sidx[j * KD + k], semi[j])
            pltpu.async_copy(dst_hbm.at[sl], didx[j * KD + k], semi[j])

    def drain_idx(off, j):
        for k in range(KD):
            sl = pl.ds(off + k * CH, CH)
            pltpu.make_async_copy(src_hbm.at[sl], sidx[j * KD + k],
                                  semi[j]).wait()
            pltpu.make_async_copy(dst_hbm.at[sl], didx[j * KD + k],
                                  semi[j]).wait()

    def fire_adds(j, h):
        for k in range(KD):
            pltpu.async_copy(ohs_v, deg_sh.at[sidx[j * KD + k]], sems[h],
                             add=True)
            pltpu.async_copy(ohd_v, deg_sh.at[didx[j * KD + k]], sems[h],
                             add=True)

    def drain_adds(j, h):
        for k in range(KD):
            pltpu.make_async_copy(ohs_v, deg_sh.at[sidx[j * KD + k]],
                                  sems[h]).wait()
            pltpu.make_async_copy(ohd_v, deg_sh.at[didx[j * KD + k]],
                                  sems[h]).wait()

    def set_off(t):
        return base + t * (KD * CH)

    fire_idx(set_off(0), 0)
    fire_idx(set_off(1), 1)
    zcp.wait()
    plsc.subcore_barrier()

    drain_idx(set_off(0), 0)
    fire_idx(set_off(2), 2)
    fire_adds(0, 0)
    drain_idx(set_off(1), 1)
    fire_idx(set_off(3), 3)
    fire_adds(1, 1)

    def body(i, carry):
        for u in range(4):
            t = 4 * i + 2 + u
            j = (2 + u) % 4
            h = u % 2
            drain_adds(j, h)
            fire_idx(set_off(t + 2), u)
            drain_idx(set_off(t), j)
            fire_adds(j, h)
        return carry

    lax.fori_loop(0, (NSETD - 6) // 4, body, 0)
    drain_adds(2, 0)
    fire_idx(set_off(60), 0)
    drain_idx(set_off(58), 2)
    fire_adds(2, 0)
    drain_adds(3, 1)
    fire_idx(set_off(61), 1)
    drain_idx(set_off(59), 3)
    fire_adds(3, 1)
    drain_adds(0, 0)
    drain_idx(set_off(60), 0)
    fire_adds(0, 0)
    drain_adds(1, 1)
    drain_idx(set_off(61), 1)
    fire_adds(1, 1)
    drain_adds(0, 0)
    drain_adds(1, 1)
    toff = base + NSETD * (KD * CH)
    pltpu.sync_copy(src_hbm.at[pl.ds(toff, CH)], sidx[0])
    pltpu.sync_copy(dst_hbm.at[pl.ds(toff, CH)], didx[0])
    pltpu.sync_copy(ohs_v, deg_sh.at[sidx[0]], add=True)
    pltpu.sync_copy(ohd_v, deg_sh.at[didx[0]], add=True)
    plsc.subcore_barrier()
    pltpu.sync_copy(deg_sh.at[rows], out_hbm.at[c, rows])


NSET = NCH // KS
PR = 4


@functools.partial(
    pl.kernel,
    out_type=jax.ShapeDtypeStruct((NC, NP, D), jnp.float32),
    mesh=_mesh,
    scratch_types=(
        [pltpu.VMEM((CH,), jnp.int32)] * (PR * KS)
        + [pltpu.VMEM((CH,), jnp.int32)] * (PR * KS)
        + [pltpu.VMEM((CH, D), jnp.float32)] * (2 * KS)
        + [
            pltpu.VMEM_SHARED((NP, D), jnp.float32),
        ]
        + [pltpu.SemaphoreType.DMA] * PR
        + [
            pltpu.SemaphoreType.DMA,
            pltpu.SemaphoreType.DMA,
            pltpu.SemaphoreType.DMA,
            pltpu.SemaphoreType.DMA,
        ]
    ),
)
def _scatter_kernel(y_hbm, src_hbm, dst_hbm, zeros_hbm, out_hbm, *rest):
    sidx = rest[0:PR * KS]
    didx = rest[PR * KS:2 * PR * KS]
    rows_v = rest[2 * PR * KS:2 * PR * KS + 2 * KS]
    agg_sh = rest[2 * PR * KS + 2 * KS]
    semi = rest[2 * PR * KS + 2 * KS + 1:2 * PR * KS + 2 * KS + 1 + PR]
    semg, sems_a, sems_b, semz = rest[2 * PR * KS + 2 * KS + 1 + PR:]
    sems = (sems_a, sems_b)
    c = lax.axis_index("c")
    s = lax.axis_index("s")
    base = (s * NC + c) * EPT
    rows = pl.ds(s * RPT, RPT)
    zcp = pltpu.async_copy(zeros_hbm, agg_sh.at[rows], semz)

    def fire_idx(off, j):
        for k in range(KS):
            sl = pl.ds(off + k * CH, CH)
            pltpu.async_copy(src_hbm.at[sl], sidx[j * KS + k], semi[j])
            pltpu.async_copy(dst_hbm.at[sl], didx[j * KS + k], semi[j])

    def drain_idx(off, j):
        for k in range(KS):
            sl = pl.ds(off + k * CH, CH)
            pltpu.make_async_copy(src_hbm.at[sl], sidx[j * KS + k],
                                  semi[j]).wait()
            pltpu.make_async_copy(dst_hbm.at[sl], didx[j * KS + k],
                                  semi[j]).wait()

    def gathers(j, h):
        ds = [
            pltpu.async_copy(y_hbm.at[sidx[j * KS + k]], rows_v[h * KS + k],
                             semg)
            for k in range(KS)
        ]
        for d in ds:
            d.wait()

    def fire_adds(j, h):
        for k in range(KS):
            pltpu.async_copy(rows_v[h * KS + k], agg_sh.at[didx[j * KS + k]],
                             sems[h], add=True)

    def drain_adds(j, h):
        for k in range(KS):
            pltpu.make_async_copy(rows_v[h * KS + k],
                                  agg_sh.at[didx[j * KS + k]],
                                  sems[h]).wait()

    def set_off(t):
        return base + t * (KS * CH)

    fire_idx(set_off(0), 0)
    fire_idx(set_off(1), 1)
    zcp.wait()
    plsc.subcore_barrier()

    drain_idx(set_off(0), 0)
    fire_idx(set_off(2), 2)
    gathers(0, 0)
    fire_adds(0, 0)
    drain_idx(set_off(1), 1)
    fire_idx(set_off(3), 3)
    gathers(1, 1)
    fire_adds(1, 1)

    def body(i, carry):
        for u in range(4):
            t = 4 * i + 2 + u
            j = (2 + u) % PR
            h = u % 2
            drain_adds(j, h)
            fire_idx(set_off(t + 2), u)
            drain_idx(set_off(t), j)
            gathers(j, h)
            fire_adds(j, h)
        return carry

    lax.fori_loop(0, (NSET - 6) // 4, body, 0)
    drain_adds(2, 0)
    fire_idx(set_off(60), 0)
    drain_idx(set_off(58), 2)
    gathers(2, 0)
    fire_adds(2, 0)
    drain_adds(3, 1)
    fire_idx(set_off(61), 1)
    drain_idx(set_off(59), 3)
    gathers(3, 1)
    fire_adds(3, 1)
    drain_adds(0, 0)
    drain_idx(set_off(60), 0)
    gathers(0, 0)
    fire_adds(0, 0)
    drain_adds(1, 1)
    drain_idx(set_off(61), 1)
    gathers(1, 1)
    fire_adds(1, 1)
    drain_adds(0, 0)
    drain_adds(1, 1)
    toff = base + NSET * (KS * CH)
    pltpu.sync_copy(src_hbm.at[pl.ds(toff, CH)], sidx[0])
    pltpu.async_copy(y_hbm.at[sidx[0]], rows_v[0], semg).wait()
    pltpu.sync_copy(dst_hbm.at[pl.ds(toff, CH)], didx[0])
    pltpu.sync_copy(rows_v[0], agg_sh.at[didx[0]], add=True)
    plsc.subcore_barrier()
    pltpu.sync_copy(agg_sh.at[rows], out_hbm.at[c, rows])


NB = 10
BR = N // NB


def _tc1_body(parts_ref, x_ref, w_ref, y_ref, nrm_ref):
    p = parts_ref[0] + parts_ref[1]
    t_out = p[:, 0:16]
    t_in = p[:, 16:32]
    deg_out = jnp.maximum(jnp.sum(t_out, axis=1, keepdims=True), 1.0)
    deg_in = jnp.maximum(jnp.sum(t_in, axis=1, keepdims=True), 1.0)
    ns = lax.rsqrt(deg_out)
    nd = lax.rsqrt(deg_in)
    nrm_ref[:, 0:1] = ns
    nrm_ref[:, 1:2] = nd
    hw = jnp.dot(x_ref[...], w_ref[...], preferred_element_type=jnp.float32)
    y_ref[...] = hw * ns


def _tc2_body(agg_ref, nrm_ref, b_ref, w_ref, y_ref):
    agg = agg_ref[0] + agg_ref[1]
    ns = nrm_ref[:, 0:1]
    nd = nrm_ref[:, 1:2]
    h = jnp.maximum(agg * nd + b_ref[...][None, :], 0.0)
    hw = jnp.dot(h, w_ref[...], preferred_element_type=jnp.float32)
    y_ref[...] = hw * ns


def _tc3_body(agg_ref, nrm_ref, b_ref, out_ref):
    agg = agg_ref[0] + agg_ref[1]
    nd = nrm_ref[:, 1:2]
    out_ref[...] = agg * nd + b_ref[...][None, :]


_row_spec = pl.BlockSpec((BR, D), lambda i: (i, 0))
_nrm_spec = pl.BlockSpec((BR, 2), lambda i: (i, 0))
_agg_spec = pl.BlockSpec((NC, BR, D), lambda i: (0, i, 0))
_w_spec = pl.BlockSpec((D, D), lambda i: (0, 0))
_b_spec = pl.BlockSpec((D,), lambda i: (0,))

_tc1 = pl.pallas_call(
    _tc1_body,
    grid=(NB,),
    in_specs=[_agg_spec, _row_spec, _w_spec],
    out_specs=(_row_spec, _nrm_spec),
    out_shape=(
        jax.ShapeDtypeStruct((N, D), jnp.float32),
        jax.ShapeDtypeStruct((N, 2), jnp.float32),
    ),
)

_tc2 = pl.pallas_call(
    _tc2_body,
    grid=(NB,),
    in_specs=[_agg_spec, _nrm_spec, _b_spec, _w_spec],
    out_specs=_row_spec,
    out_shape=jax.ShapeDtypeStruct((N, D), jnp.float32),
)

_tc3 = pl.pallas_call(
    _tc3_body,
    grid=(NB,),
    in_specs=[_agg_spec, _nrm_spec, _b_spec],
    out_specs=_row_spec,
    out_shape=jax.ShapeDtypeStruct((N, D), jnp.float32),
)


@jax.jit
def kernel(in_feat, edge_index, W1, b1, W2, b2):
    src = edge_index[0]
    dst = edge_index[1]
    eye = jnp.tile(jnp.eye(16, dtype=jnp.float32), (CH // 16, 1))
    oh_src = jnp.pad(eye, ((0, 0), (0, D - 16)))
    oh_dst = jnp.pad(eye, ((0, 0), (16, D - 32)))
    onehot = jnp.stack([oh_src, oh_dst])
    zerosd = jnp.zeros((RPT, D), jnp.float32)

    parts = _deg_kernel(src, dst, onehot, zerosd)
    y1, nrm = _tc1(parts, in_feat, W1)
    agg1 = _scatter_kernel(y1, src, dst, zerosd)
    y2 = _tc2(agg1, nrm, b1, W2)
    agg2 = _scatter_kernel(y2, src, dst, zerosd)
    return _tc3(agg2, nrm, b2)
